# Initial kernel scaffold; baseline (speedup 1.0000x reference)
import jax
import jax.numpy as jnp
from jax.experimental import pallas as pl


def kernel(x, edge_index, W_lin, b_lin, gamma2, beta2):
    raise NotImplementedError("write your pallas kernel here")


# speedup vs baseline: 8.1058x; 8.1058x over previous
import functools

import jax
import jax.numpy as jnp
from jax import lax
from jax.experimental import pallas as pl
from jax.experimental.pallas import tpu as pltpu
from jax.experimental.pallas import tpu_sc as plsc

N_NODES = 10000
N_EDGES = 320000
D = 128

NC = 2
NS = 16
L = 16
NW = NC * NS
EPW = N_EDGES // NW
G = 80
NCHUNK = EPW // G
NP = 10240
DUMMY = N_NODES
RPT = NP // NS
CW = 16


def _sc_scatter_kernel(row_hbm, col_hbm, x_hbm, acc_out, cnt_out,
                       acc_sh, cnt_sh, rowi, coli, cole, rows, ones,
                       zbuf, zbufc, sem):
    c = lax.axis_index("c")
    s = lax.axis_index("s")
    wid = c * NS + s

    def _fill_zc(i, carry):
        zbufc[i, :] = jnp.zeros((L,), jnp.float32)
        return carry
    lax.fori_loop(0, RPT, _fill_zc, 0)

    def _fill_z(i, carry):
        for j in range(D // L):
            zbuf[i, pl.ds(j * L, L)] = jnp.zeros((L,), jnp.float32)
        return carry
    lax.fori_loop(0, 64, _fill_z, 0)

    def _fill_one(i, carry):
        ones[i, :] = jnp.ones((L,), jnp.float32)
        return carry
    lax.fori_loop(0, G, _fill_one, 0)

    base_r = s * RPT
    for k in range(RPT // 64):
        pltpu.sync_copy(zbuf, acc_sh.at[pl.ds(base_r + k * 64, 64)])
    pltpu.sync_copy(zbufc, cnt_sh.at[pl.ds(base_r, RPT)])
    plsc.subcore_barrier()

    ebase = wid * EPW

    def _chunk(i, carry):
        off = ebase + i * G
        pltpu.sync_copy(row_hbm.at[pl.ds(off, G)], rowi)
        pltpu.sync_copy(col_hbm.at[pl.ds(off, G)], coli)
        for j in range(G // L):
            rv = rowi[pl.ds(j * L, L)]
            cv = coli[pl.ds(j * L, L)]
            cole[pl.ds(j * L, L)] = jnp.where(rv == cv, jnp.int32(DUMMY), cv)
        pltpu.async_copy(x_hbm.at[rowi], rows, sem).wait()
        pltpu.sync_copy(rows, acc_sh.at[cole], add=True)
        pltpu.sync_copy(ones, cnt_sh.at[cole], add=True)
        return carry
    lax.fori_loop(0, NCHUNK, _chunk, 0)

    plsc.subcore_barrier()

    out_base = c * NP + base_r
    pltpu.sync_copy(acc_sh.at[pl.ds(base_r, RPT)], acc_out.at[pl.ds(out_base, RPT)])
    pltpu.sync_copy(cnt_sh.at[pl.ds(base_r, RPT)], cnt_out.at[pl.ds(out_base, RPT)])


_sc_scatter = functools.partial(
    pl.kernel,
    out_type=(
        jax.ShapeDtypeStruct((NC * NP, D), jnp.float32),
        jax.ShapeDtypeStruct((NC * NP, CW), jnp.float32),
    ),
    mesh=plsc.VectorSubcoreMesh(core_axis_name="c", subcore_axis_name="s"),
    scratch_types=[
        pltpu.VMEM_SHARED((NP, D), jnp.float32),
        pltpu.VMEM_SHARED((NP, CW), jnp.float32),
        pltpu.VMEM((G,), jnp.int32),
        pltpu.VMEM((G,), jnp.int32),
        pltpu.VMEM((G,), jnp.int32),
        pltpu.VMEM((G, D), jnp.float32),
        pltpu.VMEM((G, CW), jnp.float32),
        pltpu.VMEM((64, D), jnp.float32),
        pltpu.VMEM((RPT, CW), jnp.float32),
        pltpu.SemaphoreType.DMA,
    ],
    compiler_params=pltpu.CompilerParams(use_tc_tiling_on_sc=False),
)(_sc_scatter_kernel)


def _tc_finish_kernel(acc_ref, cnt_ref, x_ref, w_ref, b_ref, g_ref, be_ref,
                      o_ref):
    acc = acc_ref[...]
    cnt = cnt_ref[...]
    x = x_ref[...]
    s_tot = acc[0:N_NODES] + acc[NP:NP + N_NODES] + x
    c_tot = cnt[0:N_NODES, 0:1] + cnt[NP:NP + N_NODES, 0:1] + 1.0
    aggr = s_tot / c_tot
    h = lax.dot_general(aggr, w_ref[...], (((1,), (1,)), ((), ())),
                        preferred_element_type=jnp.float32,
                        precision=lax.Precision.HIGHEST)
    h = h + b_ref[...]
    mean = jnp.mean(h, axis=0, keepdims=True)
    var = jnp.mean(jnp.square(h - mean), axis=0, keepdims=True)
    out = (h - mean) * lax.rsqrt(var + 1e-5) * g_ref[...] + be_ref[...] + x
    o_ref[...] = jnp.maximum(out, 0.0)


def _tc_finish(acc, cnt, x, W_lin, b_lin, gamma2, beta2):
    return pl.pallas_call(
        _tc_finish_kernel,
        out_shape=jax.ShapeDtypeStruct((N_NODES, D), jnp.float32),
    )(acc, cnt, x, W_lin, b_lin, gamma2, beta2)


def kernel(x, edge_index, W_lin, b_lin, gamma2, beta2):
    row = edge_index[0]
    col = edge_index[1]
    acc, cnt = _sc_scatter(row, col, x)
    return _tc_finish(acc, cnt, x, W_lin,
                      b_lin.reshape(1, D), gamma2.reshape(1, D),
                      beta2.reshape(1, D))


# speedup vs baseline: 11.0526x; 1.3635x over previous
import functools

import jax
import jax.numpy as jnp
from jax import lax
from jax.experimental import pallas as pl
from jax.experimental.pallas import tpu as pltpu
from jax.experimental.pallas import tpu_sc as plsc

N_NODES = 10000
N_EDGES = 320000
D = 128

NC = 2
NS = 16
L = 16
NW = NC * NS
EPW = N_EDGES // NW
G = 80
NCHUNK = EPW // G
NP = 10016
DUMMY = N_NODES
RPT = NP // NS
CW = 16


def _sc_scatter_kernel(row_hbm, col_hbm, x_hbm, acc_out, cnt_out,
                       acc_sh, cnt_sh, rowi0, rowi1, coli0, coli1,
                       cole0, cole1, rows0, rows1, ones, zbuf, zbufc, sem, sem2):
    c = lax.axis_index("c")
    s = lax.axis_index("s")
    wid = c * NS + s

    def _fill_zc(i, carry):
        zbufc[i, :] = jnp.zeros((L,), jnp.float32)
        return carry
    lax.fori_loop(0, RPT, _fill_zc, 0)

    def _fill_z(i, carry):
        for j in range(D // L):
            zbuf[i, pl.ds(j * L, L)] = jnp.zeros((L,), jnp.float32)
        return carry
    lax.fori_loop(0, 64, _fill_z, 0)

    def _fill_one(i, carry):
        ones[i, :] = jnp.ones((L,), jnp.float32)
        return carry
    lax.fori_loop(0, G, _fill_one, 0)

    base_r = s * RPT
    for k in range(RPT // 64):
        pltpu.sync_copy(zbuf, acc_sh.at[pl.ds(base_r + k * 64, 64)])
    _rem = RPT % 64
    if _rem:
        pltpu.sync_copy(zbuf.at[pl.ds(0, _rem)],
                        acc_sh.at[pl.ds(base_r + (RPT // 64) * 64, _rem)])
    pltpu.sync_copy(zbufc, cnt_sh.at[pl.ds(base_r, RPT)])
    plsc.subcore_barrier()

    ebase = wid * EPW
    rowi = (rowi0, rowi1)
    coli = (coli0, coli1)
    cole = (cole0, cole1)
    rows = (rows0, rows1)

    def _load_idx(i, b):
        pltpu.sync_copy(row_hbm.at[pl.ds(ebase + i * G, G)], rowi[b])
        pltpu.sync_copy(col_hbm.at[pl.ds(ebase + i * G, G)], coli[b])
        for j in range(G // L):
            rv = rowi[b][pl.ds(j * L, L)]
            cv = coli[b][pl.ds(j * L, L)]
            cole[b][pl.ds(j * L, L)] = jnp.where(
                rv == cv, jnp.int32(DUMMY), cv)

    def _main(g, carry):
        i0 = 2 * g
        _load_idx(i0, 0)
        g0 = pltpu.async_copy(x_hbm.at[rowi0], rows0, sem)
        _load_idx(i0 + 1, 1)
        g1 = pltpu.async_copy(x_hbm.at[rowi1], rows1, sem2)
        pltpu.sync_copy(ones, cnt_sh.at[cole0], add=True)
        pltpu.sync_copy(ones, cnt_sh.at[cole1], add=True)
        g0.wait()
        pltpu.sync_copy(rows0, acc_sh.at[cole0], add=True)
        g1.wait()
        pltpu.sync_copy(rows1, acc_sh.at[cole1], add=True)
        return carry
    lax.fori_loop(0, NCHUNK // 2, _main, 0)

    last = NCHUNK - 1
    _load_idx(last, 0)
    pltpu.async_copy(x_hbm.at[rowi0], rows0, sem).wait()
    pltpu.sync_copy(rows0, acc_sh.at[cole0], add=True)
    pltpu.sync_copy(ones, cnt_sh.at[cole0], add=True)

    plsc.subcore_barrier()

    out_base = c * NP + base_r
    pltpu.sync_copy(acc_sh.at[pl.ds(base_r, RPT)],
                    acc_out.at[pl.ds(out_base, RPT)])
    pltpu.sync_copy(cnt_sh.at[pl.ds(base_r, RPT)],
                    cnt_out.at[pl.ds(out_base, RPT)])


_sc_scatter = functools.partial(
    pl.kernel,
    out_type=(
        jax.ShapeDtypeStruct((NC * NP, D), jnp.float32),
        jax.ShapeDtypeStruct((NC * NP, CW), jnp.float32),
    ),
    mesh=plsc.VectorSubcoreMesh(core_axis_name="c", subcore_axis_name="s"),
    scratch_types=[
        pltpu.VMEM_SHARED((NP, D), jnp.float32),
        pltpu.VMEM_SHARED((NP, CW), jnp.float32),
        pltpu.VMEM((G,), jnp.int32),
        pltpu.VMEM((G,), jnp.int32),
        pltpu.VMEM((G,), jnp.int32),
        pltpu.VMEM((G,), jnp.int32),
        pltpu.VMEM((G,), jnp.int32),
        pltpu.VMEM((G,), jnp.int32),
        pltpu.VMEM((G, D), jnp.float32),
        pltpu.VMEM((G, D), jnp.float32),
        pltpu.VMEM((G, CW), jnp.float32),
        pltpu.VMEM((64, D), jnp.float32),
        pltpu.VMEM((RPT, CW), jnp.float32),
        pltpu.SemaphoreType.DMA,
        pltpu.SemaphoreType.DMA,
    ],
    compiler_params=pltpu.CompilerParams(use_tc_tiling_on_sc=False),
)(_sc_scatter_kernel)


def _tc_finish_kernel(acc_ref, cnt_ref, x_ref, w_ref, b_ref, g_ref, be_ref,
                      o_ref):
    acc = acc_ref[...]
    cnt = cnt_ref[...]
    x = x_ref[...]
    s_tot = acc[0:N_NODES] + acc[NP:NP + N_NODES] + x
    c_tot = cnt[0:N_NODES, 0:1] + cnt[NP:NP + N_NODES, 0:1] + 1.0
    aggr = s_tot / c_tot
    h = lax.dot_general(aggr, w_ref[...], (((1,), (1,)), ((), ())),
                        preferred_element_type=jnp.float32,
                        precision=lax.Precision.HIGHEST)
    h = h + b_ref[...]
    mean = jnp.mean(h, axis=0, keepdims=True)
    var = jnp.mean(jnp.square(h - mean), axis=0, keepdims=True)
    out = (h - mean) * lax.rsqrt(var + 1e-5) * g_ref[...] + be_ref[...] + x
    o_ref[...] = jnp.maximum(out, 0.0)


def _tc_finish(acc, cnt, x, W_lin, b_lin, gamma2, beta2):
    return pl.pallas_call(
        _tc_finish_kernel,
        out_shape=jax.ShapeDtypeStruct((N_NODES, D), jnp.float32),
    )(acc, cnt, x, W_lin, b_lin, gamma2, beta2)


def kernel(x, edge_index, W_lin, b_lin, gamma2, beta2):
    row = edge_index[0]
    col = edge_index[1]
    acc, cnt = _sc_scatter(row, col, x)
    return _tc_finish(acc, cnt, x, W_lin,
                      b_lin.reshape(1, D), gamma2.reshape(1, D),
                      beta2.reshape(1, D))


# speedup vs baseline: 11.1565x; 1.0094x over previous
import functools

import jax
import jax.numpy as jnp
from jax import lax
from jax.experimental import pallas as pl
from jax.experimental.pallas import tpu as pltpu
from jax.experimental.pallas import tpu_sc as plsc

N_NODES = 10000
N_EDGES = 320000
D = 128

NC = 2
NS = 16
L = 16
NW = NC * NS
EPW = N_EDGES // NW
G = 80
NCHUNK = EPW // G
NP = 10016
DUMMY = N_NODES
RPT = NP // NS
CW = 16


def _sc_scatter_kernel(row_hbm, col_hbm, x_hbm, acc_out, cnt_out,
                       acc_sh, cnt_sh, rowi0, rowi1, coli0, coli1,
                       cole0, cole1, rows0, rows1, ones, zbuf, zbufc, sem, sem2,
                       sem3, sem4):
    c = lax.axis_index("c")
    s = lax.axis_index("s")
    wid = c * NS + s

    def _fill_zc(i, carry):
        zbufc[i, :] = jnp.zeros((L,), jnp.float32)
        return carry
    lax.fori_loop(0, 64, _fill_zc, 0)

    def _fill_z(i, carry):
        for j in range(D // L):
            zbuf[i, pl.ds(j * L, L)] = jnp.zeros((L,), jnp.float32)
        return carry
    lax.fori_loop(0, 64, _fill_z, 0)

    def _fill_one(i, carry):
        ones[i, :] = jnp.ones((L,), jnp.float32)
        return carry
    lax.fori_loop(0, G, _fill_one, 0)

    base_r = s * RPT
    for k in range(RPT // 64):
        pltpu.sync_copy(zbuf, acc_sh.at[pl.ds(base_r + k * 64, 64)])
    _rem = RPT % 64
    if _rem:
        pltpu.sync_copy(zbuf.at[pl.ds(0, _rem)],
                        acc_sh.at[pl.ds(base_r + (RPT // 64) * 64, _rem)])
    for k in range(RPT // 64):
        pltpu.sync_copy(zbufc, cnt_sh.at[pl.ds(base_r + k * 64, 64)])
    if _rem:
        pltpu.sync_copy(zbufc.at[pl.ds(0, _rem)],
                        cnt_sh.at[pl.ds(base_r + (RPT // 64) * 64, _rem)])
    plsc.subcore_barrier()

    ebase = wid * EPW
    rowi = (rowi0, rowi1)
    coli = (coli0, coli1)
    cole = (cole0, cole1)
    rows = (rows0, rows1)

    def _load_idx(i, b):
        pltpu.sync_copy(row_hbm.at[pl.ds(ebase + i * G, G)], rowi[b])
        pltpu.sync_copy(col_hbm.at[pl.ds(ebase + i * G, G)], coli[b])
        for j in range(G // L):
            rv = rowi[b][pl.ds(j * L, L)]
            cv = coli[b][pl.ds(j * L, L)]
            cole[b][pl.ds(j * L, L)] = jnp.where(
                rv == cv, jnp.int32(DUMMY), cv)

    def _main(g, carry):
        i0 = 2 * g
        _load_idx(i0, 0)
        g0 = pltpu.async_copy(x_hbm.at[rowi0], rows0, sem)
        _load_idx(i0 + 1, 1)
        g1 = pltpu.async_copy(x_hbm.at[rowi1], rows1, sem2)
        c0 = pltpu.async_copy(ones, cnt_sh.at[cole0], sem3, add=True)
        c1 = pltpu.async_copy(ones, cnt_sh.at[cole1], sem3, add=True)
        g0.wait()
        a0 = pltpu.async_copy(rows0, acc_sh.at[cole0], sem4, add=True)
        g1.wait()
        a1 = pltpu.async_copy(rows1, acc_sh.at[cole1], sem4, add=True)
        c0.wait()
        c1.wait()
        a0.wait()
        a1.wait()
        return carry
    lax.fori_loop(0, NCHUNK // 2, _main, 0)

    last = NCHUNK - 1
    _load_idx(last, 0)
    pltpu.async_copy(x_hbm.at[rowi0], rows0, sem).wait()
    pltpu.sync_copy(rows0, acc_sh.at[cole0], add=True)
    pltpu.sync_copy(ones, cnt_sh.at[cole0], add=True)

    plsc.subcore_barrier()

    out_base = c * NP + base_r
    pltpu.sync_copy(acc_sh.at[pl.ds(base_r, RPT)],
                    acc_out.at[pl.ds(out_base, RPT)])
    pltpu.sync_copy(cnt_sh.at[pl.ds(base_r, RPT)],
                    cnt_out.at[pl.ds(out_base, RPT)])


_sc_scatter = functools.partial(
    pl.kernel,
    out_type=(
        jax.ShapeDtypeStruct((NC * NP, D), jnp.float32),
        jax.ShapeDtypeStruct((NC * NP, CW), jnp.float32),
    ),
    mesh=plsc.VectorSubcoreMesh(core_axis_name="c", subcore_axis_name="s"),
    scratch_types=[
        pltpu.VMEM_SHARED((NP, D), jnp.float32),
        pltpu.VMEM_SHARED((NP, CW), jnp.float32),
        pltpu.VMEM((G,), jnp.int32),
        pltpu.VMEM((G,), jnp.int32),
        pltpu.VMEM((G,), jnp.int32),
        pltpu.VMEM((G,), jnp.int32),
        pltpu.VMEM((G,), jnp.int32),
        pltpu.VMEM((G,), jnp.int32),
        pltpu.VMEM((G, D), jnp.float32),
        pltpu.VMEM((G, D), jnp.float32),
        pltpu.VMEM((G, CW), jnp.float32),
        pltpu.VMEM((64, D), jnp.float32),
        pltpu.VMEM((64, CW), jnp.float32),
        pltpu.SemaphoreType.DMA,
        pltpu.SemaphoreType.DMA,
        pltpu.SemaphoreType.DMA,
        pltpu.SemaphoreType.DMA,
    ],
    compiler_params=pltpu.CompilerParams(use_tc_tiling_on_sc=False),
)(_sc_scatter_kernel)


def _tc_finish_kernel(acc_ref, cnt_ref, x_ref, w_ref, b_ref, g_ref, be_ref,
                      o_ref):
    acc = acc_ref[...]
    cnt = cnt_ref[...]
    x = x_ref[...]
    s_tot = acc[0:N_NODES] + acc[NP:NP + N_NODES] + x
    c_tot = cnt[0:N_NODES, 0:1] + cnt[NP:NP + N_NODES, 0:1] + 1.0
    aggr = s_tot / c_tot
    h = lax.dot_general(aggr, w_ref[...], (((1,), (1,)), ((), ())),
                        preferred_element_type=jnp.float32,
                        precision=lax.Precision.HIGHEST)
    h = h + b_ref[...]
    mean = jnp.mean(h, axis=0, keepdims=True)
    var = jnp.mean(jnp.square(h - mean), axis=0, keepdims=True)
    out = (h - mean) * lax.rsqrt(var + 1e-5) * g_ref[...] + be_ref[...] + x
    o_ref[...] = jnp.maximum(out, 0.0)


def _tc_finish(acc, cnt, x, W_lin, b_lin, gamma2, beta2):
    return pl.pallas_call(
        _tc_finish_kernel,
        out_shape=jax.ShapeDtypeStruct((N_NODES, D), jnp.float32),
    )(acc, cnt, x, W_lin, b_lin, gamma2, beta2)


def kernel(x, edge_index, W_lin, b_lin, gamma2, beta2):
    row = edge_index[0]
    col = edge_index[1]
    acc, cnt = _sc_scatter(row, col, x)
    return _tc_finish(acc, cnt, x, W_lin,
                      b_lin.reshape(1, D), gamma2.reshape(1, D),
                      beta2.reshape(1, D))


# speedup vs baseline: 11.3893x; 1.0209x over previous
import functools

import jax
import jax.numpy as jnp
from jax import lax
from jax.experimental import pallas as pl
from jax.experimental.pallas import tpu as pltpu
from jax.experimental.pallas import tpu_sc as plsc

N_NODES = 10000
N_EDGES = 320000
D = 128

NC = 2
NS = 16
L = 16
NW = NC * NS
EPW = N_EDGES // NW
G = 80
NCHUNK = EPW // G
NP = 10016
DUMMY = N_NODES
RPT = NP // NS
CW = 16


def _sc_scatter_kernel(row_hbm, col_hbm, x_hbm, acc_out, cnt_out,
                       acc_sh, cnt_sh, rowi0, rowi1, coli0, coli1,
                       cole0, cole1, rows0, rows1, ones, zbuf, zbufc, sem, sem2,
                       sem3, sem4):
    c = lax.axis_index("c")
    s = lax.axis_index("s")
    wid = c * NS + s

    def _fill_zc(i, carry):
        zbufc[i, :] = jnp.zeros((L,), jnp.float32)
        return carry
    lax.fori_loop(0, 64, _fill_zc, 0)

    def _fill_z(i, carry):
        for j in range(D // (2 * L)):
            zbuf[i, pl.ds(j * 2 * L, 2 * L)] = jnp.zeros((2 * L,),
                                                         jnp.bfloat16)
        return carry
    lax.fori_loop(0, 64, _fill_z, 0)

    def _fill_one(i, carry):
        ones[i, :] = jnp.ones((L,), jnp.float32)
        return carry
    lax.fori_loop(0, G, _fill_one, 0)

    base_r = s * RPT
    for k in range(RPT // 64):
        pltpu.sync_copy(zbuf, acc_sh.at[pl.ds(base_r + k * 64, 64)])
    _rem = RPT % 64
    if _rem:
        pltpu.sync_copy(zbuf.at[pl.ds(0, _rem)],
                        acc_sh.at[pl.ds(base_r + (RPT // 64) * 64, _rem)])
    for k in range(RPT // 64):
        pltpu.sync_copy(zbufc, cnt_sh.at[pl.ds(base_r + k * 64, 64)])
    if _rem:
        pltpu.sync_copy(zbufc.at[pl.ds(0, _rem)],
                        cnt_sh.at[pl.ds(base_r + (RPT // 64) * 64, _rem)])
    plsc.subcore_barrier()

    ebase = wid * EPW
    rowi = (rowi0, rowi1)
    coli = (coli0, coli1)
    cole = (cole0, cole1)
    rows = (rows0, rows1)

    def _load_idx(i, b):
        pltpu.sync_copy(row_hbm.at[pl.ds(ebase + i * G, G)], rowi[b])
        pltpu.sync_copy(col_hbm.at[pl.ds(ebase + i * G, G)], coli[b])
        for j in range(G // L):
            rv = rowi[b][pl.ds(j * L, L)]
            cv = coli[b][pl.ds(j * L, L)]
            cole[b][pl.ds(j * L, L)] = jnp.where(
                rv == cv, jnp.int32(DUMMY), cv)

    def _main(g, carry):
        i0 = 2 * g
        _load_idx(i0, 0)
        g0 = pltpu.async_copy(x_hbm.at[rowi0], rows0, sem)
        _load_idx(i0 + 1, 1)
        g1 = pltpu.async_copy(x_hbm.at[rowi1], rows1, sem2)
        c0 = pltpu.async_copy(ones, cnt_sh.at[cole0], sem3, add=True)
        c1 = pltpu.async_copy(ones, cnt_sh.at[cole1], sem3, add=True)
        g0.wait()
        a0 = pltpu.async_copy(rows0, acc_sh.at[cole0], sem4, add=True)
        g1.wait()
        a1 = pltpu.async_copy(rows1, acc_sh.at[cole1], sem4, add=True)
        c0.wait()
        c1.wait()
        a0.wait()
        a1.wait()
        return carry
    lax.fori_loop(0, NCHUNK // 2, _main, 0)

    last = NCHUNK - 1
    _load_idx(last, 0)
    pltpu.async_copy(x_hbm.at[rowi0], rows0, sem).wait()
    pltpu.sync_copy(rows0, acc_sh.at[cole0], add=True)
    pltpu.sync_copy(ones, cnt_sh.at[cole0], add=True)

    plsc.subcore_barrier()

    out_base = c * NP + base_r
    pltpu.sync_copy(acc_sh.at[pl.ds(base_r, RPT)],
                    acc_out.at[pl.ds(out_base, RPT)])
    pltpu.sync_copy(cnt_sh.at[pl.ds(base_r, RPT)],
                    cnt_out.at[pl.ds(out_base, RPT)])


_sc_scatter = functools.partial(
    pl.kernel,
    out_type=(
        jax.ShapeDtypeStruct((NC * NP, D), jnp.bfloat16),
        jax.ShapeDtypeStruct((NC * NP, CW), jnp.float32),
    ),
    mesh=plsc.VectorSubcoreMesh(core_axis_name="c", subcore_axis_name="s"),
    scratch_types=[
        pltpu.VMEM_SHARED((NP, D), jnp.bfloat16),
        pltpu.VMEM_SHARED((NP, CW), jnp.float32),
        pltpu.VMEM((G,), jnp.int32),
        pltpu.VMEM((G,), jnp.int32),
        pltpu.VMEM((G,), jnp.int32),
        pltpu.VMEM((G,), jnp.int32),
        pltpu.VMEM((G,), jnp.int32),
        pltpu.VMEM((G,), jnp.int32),
        pltpu.VMEM((G, D), jnp.bfloat16),
        pltpu.VMEM((G, D), jnp.bfloat16),
        pltpu.VMEM((G, CW), jnp.float32),
        pltpu.VMEM((64, D), jnp.bfloat16),
        pltpu.VMEM((64, CW), jnp.float32),
        pltpu.SemaphoreType.DMA,
        pltpu.SemaphoreType.DMA,
        pltpu.SemaphoreType.DMA,
        pltpu.SemaphoreType.DMA,
    ],
    compiler_params=pltpu.CompilerParams(use_tc_tiling_on_sc=False),
)(_sc_scatter_kernel)


def _tc_finish_kernel(acc_ref, cnt_ref, x_ref, w_ref, b_ref, g_ref, be_ref,
                      o_ref):
    acc = acc_ref[...]
    cnt = cnt_ref[...]
    x = x_ref[...]
    accf = acc.astype(jnp.float32)
    s_tot = accf[0:N_NODES] + accf[NP:NP + N_NODES] + x
    c_tot = cnt[0:N_NODES, 0:1] + cnt[NP:NP + N_NODES, 0:1] + 1.0
    aggr = s_tot / c_tot
    h = lax.dot_general(aggr, w_ref[...], (((1,), (1,)), ((), ())),
                        preferred_element_type=jnp.float32,
                        precision=lax.Precision.HIGHEST)
    h = h + b_ref[...]
    mean = jnp.mean(h, axis=0, keepdims=True)
    var = jnp.mean(jnp.square(h - mean), axis=0, keepdims=True)
    out = (h - mean) * lax.rsqrt(var + 1e-5) * g_ref[...] + be_ref[...] + x
    o_ref[...] = jnp.maximum(out, 0.0)


def _tc_finish(acc, cnt, x, W_lin, b_lin, gamma2, beta2):
    return pl.pallas_call(
        _tc_finish_kernel,
        out_shape=jax.ShapeDtypeStruct((N_NODES, D), jnp.float32),
    )(acc, cnt, x, W_lin, b_lin, gamma2, beta2)


def kernel(x, edge_index, W_lin, b_lin, gamma2, beta2):
    row = edge_index[0]
    col = edge_index[1]
    acc, cnt = _sc_scatter(row, col, x.astype(jnp.bfloat16))
    return _tc_finish(acc, cnt, x, W_lin,
                      b_lin.reshape(1, D), gamma2.reshape(1, D),
                      beta2.reshape(1, D))


# speedup vs baseline: 14.7815x; 1.2978x over previous
import functools

import jax
import jax.numpy as jnp
from jax import lax
from jax.experimental import pallas as pl
from jax.experimental.pallas import tpu as pltpu
from jax.experimental.pallas import tpu_sc as plsc

N_NODES = 10000
N_EDGES = 320000
D = 128

NC = 2
NS = 16
L = 16
NW = NC * NS
EPW = N_EDGES // NW
G = 80
NCHUNK = EPW // G
K = 5
NP = 10016
DUMMY = N_NODES
RPT = NP // NS
CW = 16


def _sc_scatter_kernel(row_hbm, col_hbm, x_hbm, acc_out, cnt_out,
                       acc_sh, cnt_sh, rowb, colb, cole2d,
                       rows0, rows1, ones, zbuf, zbufc, sem, sem2,
                       sem3, sem4):
    c = lax.axis_index("c")
    s = lax.axis_index("s")
    wid = c * NS + s

    def _fill_zc(i, carry):
        zbufc[i, :] = jnp.zeros((L,), jnp.float32)
        return carry
    lax.fori_loop(0, 64, _fill_zc, 0)

    def _fill_z(i, carry):
        for j in range(D // (2 * L)):
            zbuf[i, pl.ds(j * 2 * L, 2 * L)] = jnp.zeros((2 * L,),
                                                         jnp.bfloat16)
        return carry
    lax.fori_loop(0, 64, _fill_z, 0)

    def _fill_one(i, carry):
        ones[i, :] = jnp.ones((L,), jnp.float32)
        return carry
    lax.fori_loop(0, G, _fill_one, 0)

    base_r = s * RPT
    for k in range(RPT // 64):
        pltpu.sync_copy(zbuf, acc_sh.at[pl.ds(base_r + k * 64, 64)])
    _rem = RPT % 64
    if _rem:
        pltpu.sync_copy(zbuf.at[pl.ds(0, _rem)],
                        acc_sh.at[pl.ds(base_r + (RPT // 64) * 64, _rem)])
    for k in range(RPT // 64):
        pltpu.sync_copy(zbufc, cnt_sh.at[pl.ds(base_r + k * 64, 64)])
    if _rem:
        pltpu.sync_copy(zbufc.at[pl.ds(0, _rem)],
                        cnt_sh.at[pl.ds(base_r + (RPT // 64) * 64, _rem)])
    plsc.subcore_barrier()

    ebase = wid * EPW
    rows = (rows0, rows1)
    gsem = (sem, sem2)

    def _macro(m, carry):
        off = ebase + m * (K * G)
        pltpu.sync_copy(row_hbm.at[pl.ds(off, K * G)], rowb)
        pltpu.sync_copy(col_hbm.at[pl.ds(off, K * G)], colb)
        for j in range(K):
            for q in range(G // L):
                rv = rowb[pl.ds(j * G + q * L, L)]
                cv = colb[pl.ds(j * G + q * L, L)]
                cole2d[j, pl.ds(q * L, L)] = jnp.where(
                    rv == cv, jnp.int32(DUMMY), cv)
        gd = [None] * K
        ad = [None] * K
        cd = [None] * K
        gd[0] = pltpu.async_copy(x_hbm.at[rowb.at[pl.ds(0, G)]], rows[0],
                                 gsem[0])
        for j in range(K):
            if j >= 1:
                ad[j - 1].wait()
                cd[j - 1].wait()
            if j + 1 < K:
                gd[j + 1] = pltpu.async_copy(
                    x_hbm.at[rowb.at[pl.ds((j + 1) * G, G)]],
                    rows[(j + 1) % 2], gsem[(j + 1) % 2])
            gd[j].wait()
            cd[j] = pltpu.async_copy(ones, cnt_sh.at[cole2d.at[j]], sem3,
                                     add=True)
            ad[j] = pltpu.async_copy(rows[j % 2], acc_sh.at[cole2d.at[j]],
                                     sem4, add=True)
        ad[K - 1].wait()
        cd[K - 1].wait()
        return carry
    lax.fori_loop(0, NCHUNK // K, _macro, 0)

    plsc.subcore_barrier()

    out_base = c * NP + base_r
    pltpu.sync_copy(acc_sh.at[pl.ds(base_r, RPT)],
                    acc_out.at[pl.ds(out_base, RPT)])
    pltpu.sync_copy(cnt_sh.at[pl.ds(base_r, RPT)],
                    cnt_out.at[pl.ds(out_base, RPT)])


_sc_scatter = functools.partial(
    pl.kernel,
    out_type=(
        jax.ShapeDtypeStruct((NC * NP, D), jnp.bfloat16),
        jax.ShapeDtypeStruct((NC * NP, CW), jnp.float32),
    ),
    mesh=plsc.VectorSubcoreMesh(core_axis_name="c", subcore_axis_name="s"),
    scratch_types=[
        pltpu.VMEM_SHARED((NP, D), jnp.bfloat16),
        pltpu.VMEM_SHARED((NP, CW), jnp.float32),
        pltpu.VMEM((K * G,), jnp.int32),
        pltpu.VMEM((K * G,), jnp.int32),
        pltpu.VMEM((K, G), jnp.int32),
        pltpu.VMEM((G, D), jnp.bfloat16),
        pltpu.VMEM((G, D), jnp.bfloat16),
        pltpu.VMEM((G, CW), jnp.float32),
        pltpu.VMEM((64, D), jnp.bfloat16),
        pltpu.VMEM((64, CW), jnp.float32),
        pltpu.SemaphoreType.DMA,
        pltpu.SemaphoreType.DMA,
        pltpu.SemaphoreType.DMA,
        pltpu.SemaphoreType.DMA,
    ],
    compiler_params=pltpu.CompilerParams(use_tc_tiling_on_sc=False),
)(_sc_scatter_kernel)


def _tc_finish_kernel(acc_ref, cnt_ref, x_ref, w_ref, b_ref, g_ref, be_ref,
                      o_ref):
    acc = acc_ref[...]
    cnt = cnt_ref[...]
    x = x_ref[...]
    accf = acc.astype(jnp.float32)
    s_tot = accf[0:N_NODES] + accf[NP:NP + N_NODES] + x
    c_tot = cnt[0:N_NODES, 0:1] + cnt[NP:NP + N_NODES, 0:1] + 1.0
    aggr = s_tot / c_tot
    h = lax.dot_general(aggr, w_ref[...], (((1,), (1,)), ((), ())),
                        preferred_element_type=jnp.float32,
                        precision=lax.Precision.HIGHEST)
    h = h + b_ref[...]
    mean = jnp.mean(h, axis=0, keepdims=True)
    var = jnp.mean(jnp.square(h - mean), axis=0, keepdims=True)
    out = (h - mean) * lax.rsqrt(var + 1e-5) * g_ref[...] + be_ref[...] + x
    o_ref[...] = jnp.maximum(out, 0.0)


def _tc_finish(acc, cnt, x, W_lin, b_lin, gamma2, beta2):
    return pl.pallas_call(
        _tc_finish_kernel,
        out_shape=jax.ShapeDtypeStruct((N_NODES, D), jnp.float32),
    )(acc, cnt, x, W_lin, b_lin, gamma2, beta2)


def kernel(x, edge_index, W_lin, b_lin, gamma2, beta2):
    row = edge_index[0]
    col = edge_index[1]
    acc, cnt = _sc_scatter(row, col, x.astype(jnp.bfloat16))
    return _tc_finish(acc, cnt, x, W_lin,
                      b_lin.reshape(1, D), gamma2.reshape(1, D),
                      beta2.reshape(1, D))


# speedup vs baseline: 15.1069x; 1.0220x over previous
import functools

import jax
import jax.numpy as jnp
from jax import lax
from jax.experimental import pallas as pl
from jax.experimental.pallas import tpu as pltpu
from jax.experimental.pallas import tpu_sc as plsc

N_NODES = 10000
N_EDGES = 320000
D = 128

NC = 2
NS = 16
L = 16
NW = NC * NS
EPW = N_EDGES // NW
G = 80
NCHUNK = EPW // G
K = 5
NP = 10240
DUMMY = N_NODES
RPT = NP // NS


def _sc_scatter_kernel(row_hbm, col_hbm, x_hbm, acc_out, cnt_out,
                       acc_sh, cnt16_sh, rowb, colb, cole2d,
                       rows0, rows1, hist, redtmp, redacc, zbuf,
                       sem, sem2, sem4):
    c = lax.axis_index("c")
    s = lax.axis_index("s")
    wid = c * NS + s

    def _fill_h(i, carry):
        hist[pl.ds(i * L, L)] = jnp.zeros((L,), jnp.float32)
        return carry
    lax.fori_loop(0, NP // L, _fill_h, 0)

    def _fill_z(i, carry):
        for j in range(D // (2 * L)):
            zbuf[i, pl.ds(j * 2 * L, 2 * L)] = jnp.zeros((2 * L,),
                                                         jnp.bfloat16)
        return carry
    lax.fori_loop(0, 64, _fill_z, 0)

    base_r = s * RPT
    for k in range(RPT // 64):
        pltpu.sync_copy(zbuf, acc_sh.at[pl.ds(base_r + k * 64, 64)])
    plsc.subcore_barrier()

    ebase = wid * EPW
    rows = (rows0, rows1)
    gsem = (sem, sem2)
    one_v = jnp.ones((L,), jnp.float32)

    def _macro(m, carry):
        off = ebase + m * (K * G)
        pltpu.sync_copy(row_hbm.at[pl.ds(off, K * G)], rowb)
        pltpu.sync_copy(col_hbm.at[pl.ds(off, K * G)], colb)
        for j in range(K):
            for q in range(G // L):
                rv = rowb[pl.ds(j * G + q * L, L)]
                cv = colb[pl.ds(j * G + q * L, L)]
                ce = jnp.where(rv == cv, jnp.int32(DUMMY), cv)
                cole2d[j, pl.ds(q * L, L)] = ce
                plsc.addupdate_scatter(hist, [ce], one_v)
        gd = [None] * K
        ad = [None] * K
        gd[0] = pltpu.async_copy(x_hbm.at[rowb.at[pl.ds(0, G)]], rows[0],
                                 gsem[0])
        for j in range(K):
            if j >= 1:
                ad[j - 1].wait()
            if j + 1 < K:
                gd[j + 1] = pltpu.async_copy(
                    x_hbm.at[rowb.at[pl.ds((j + 1) * G, G)]],
                    rows[(j + 1) % 2], gsem[(j + 1) % 2])
            gd[j].wait()
            ad[j] = pltpu.async_copy(rows[j % 2], acc_sh.at[cole2d.at[j]],
                                     sem4, add=True)
        ad[K - 1].wait()
        return carry
    lax.fori_loop(0, NCHUNK // K, _macro, 0)

    plsc.subcore_barrier()

    pltpu.sync_copy(hist, cnt16_sh.at[s])
    plsc.subcore_barrier()

    pltpu.sync_copy(cnt16_sh.at[0, pl.ds(base_r, RPT)], redacc)
    for t in range(1, NS):
        pltpu.sync_copy(cnt16_sh.at[t, pl.ds(base_r, RPT)], redtmp)
        for q in range(RPT // L):
            redacc[pl.ds(q * L, L)] = (redacc[pl.ds(q * L, L)]
                                       + redtmp[pl.ds(q * L, L)])

    out_base = c * NP + base_r
    pltpu.sync_copy(acc_sh.at[pl.ds(base_r, RPT)],
                    acc_out.at[pl.ds(out_base, RPT)])
    pltpu.sync_copy(redacc, cnt_out.at[pl.ds(out_base, RPT)])


_sc_scatter = functools.partial(
    pl.kernel,
    out_type=(
        jax.ShapeDtypeStruct((NC * NP, D), jnp.bfloat16),
        jax.ShapeDtypeStruct((NC * NP,), jnp.float32),
    ),
    mesh=plsc.VectorSubcoreMesh(core_axis_name="c", subcore_axis_name="s"),
    scratch_types=[
        pltpu.VMEM_SHARED((NP, D), jnp.bfloat16),
        pltpu.VMEM_SHARED((NS, NP), jnp.float32),
        pltpu.VMEM((K * G,), jnp.int32),
        pltpu.VMEM((K * G,), jnp.int32),
        pltpu.VMEM((K, G), jnp.int32),
        pltpu.VMEM((G, D), jnp.bfloat16),
        pltpu.VMEM((G, D), jnp.bfloat16),
        pltpu.VMEM((NP,), jnp.float32),
        pltpu.VMEM((RPT,), jnp.float32),
        pltpu.VMEM((RPT,), jnp.float32),
        pltpu.VMEM((64, D), jnp.bfloat16),
        pltpu.SemaphoreType.DMA,
        pltpu.SemaphoreType.DMA,
        pltpu.SemaphoreType.DMA,
    ],
    compiler_params=pltpu.CompilerParams(use_tc_tiling_on_sc=False,
                                         needs_layout_passes=False),
)(_sc_scatter_kernel)


def _tc_finish_kernel(acc_ref, cnt_ref, x_ref, w_ref, b_ref, g_ref, be_ref,
                      o_ref):
    acc = acc_ref[...]
    cnt = cnt_ref[...]
    x = x_ref[...]
    accf = acc.astype(jnp.float32)
    s_tot = accf[0:N_NODES] + accf[NP:NP + N_NODES] + x
    c_tot = (cnt[0:N_NODES] + cnt[NP:NP + N_NODES] + 1.0).reshape(N_NODES, 1)
    aggr = s_tot / c_tot
    h = lax.dot_general(aggr, w_ref[...], (((1,), (1,)), ((), ())),
                        preferred_element_type=jnp.float32,
                        precision=lax.Precision.HIGHEST)
    h = h + b_ref[...]
    mean = jnp.mean(h, axis=0, keepdims=True)
    var = jnp.mean(jnp.square(h - mean), axis=0, keepdims=True)
    out = (h - mean) * lax.rsqrt(var + 1e-5) * g_ref[...] + be_ref[...] + x
    o_ref[...] = jnp.maximum(out, 0.0)


def _tc_finish(acc, cnt, x, W_lin, b_lin, gamma2, beta2):
    return pl.pallas_call(
        _tc_finish_kernel,
        out_shape=jax.ShapeDtypeStruct((N_NODES, D), jnp.float32),
    )(acc, cnt, x, W_lin, b_lin, gamma2, beta2)


def kernel(x, edge_index, W_lin, b_lin, gamma2, beta2):
    row = edge_index[0]
    col = edge_index[1]
    acc, cnt = _sc_scatter(row, col, x.astype(jnp.bfloat16))
    return _tc_finish(acc, cnt, x, W_lin,
                      b_lin.reshape(1, D), gamma2.reshape(1, D),
                      beta2.reshape(1, D))


# speedup vs baseline: 17.1227x; 1.1334x over previous
import functools

import jax
import jax.numpy as jnp
from jax import lax
from jax.experimental import pallas as pl
from jax.experimental.pallas import tpu as pltpu
from jax.experimental.pallas import tpu_sc as plsc

N_NODES = 10000
N_EDGES = 320000
D = 128

NC = 2
NS = 16
L = 16
NW = NC * NS
EPW = N_EDGES // NW
G = 80
NCHUNK = EPW // G
K = 5
NP = 10240
DUMMY = N_NODES
RPT = NP // NS


def _sc_scatter_kernel(edge_hbm, x_hbm, acc_out, cnt_out,
                       acc_sh, cnt16_sh, rowb, colb, cole2d,
                       rows0, rows1, hist, redtmp, redacc, zbuf,
                       sem, sem2, sem4):
    c = lax.axis_index("c")
    s = lax.axis_index("s")
    wid = c * NS + s

    def _fill_h(i, carry):
        for u in range(4):
            hist[pl.ds((i * 4 + u) * L, L)] = jnp.zeros((L,), jnp.float32)
        return carry
    lax.fori_loop(0, NP // (4 * L), _fill_h, 0)

    def _fill_z(i, carry):
        for j in range(D // (2 * L)):
            zbuf[i, pl.ds(j * 2 * L, 2 * L)] = jnp.zeros((2 * L,),
                                                         jnp.bfloat16)
        return carry
    lax.fori_loop(0, 64, _fill_z, 0)

    base_r = s * RPT
    for k in range(RPT // 64):
        pltpu.sync_copy(zbuf, acc_sh.at[pl.ds(base_r + k * 64, 64)])
    plsc.subcore_barrier()

    ebase = wid * EPW
    rows = (rows0, rows1)
    gsem = (sem, sem2)
    one_v = jnp.ones((L,), jnp.float32)

    def _cole(j):
        for q in range(G // L):
            rv = rowb[pl.ds(j * G + q * L, L)]
            cv = colb[pl.ds(j * G + q * L, L)]
            ce = jnp.where(rv == cv, jnp.int32(DUMMY), cv)
            cole2d[j, pl.ds(q * L, L)] = ce
            plsc.addupdate_scatter(hist, [ce], one_v)

    def _macro(m, carry):
        off = ebase + m * (K * G)
        ri = pltpu.async_copy(edge_hbm.at[0, pl.ds(off, K * G)], rowb, sem)
        ci = pltpu.async_copy(edge_hbm.at[1, pl.ds(off, K * G)], colb, sem2)
        ri.wait()
        ci.wait()
        gd = [None] * K
        ad = [None] * K
        gd[0] = pltpu.async_copy(x_hbm.at[rowb.at[pl.ds(0, G)]], rows[0],
                                 gsem[0])
        _cole(0)
        for j in range(K):
            if j >= 1:
                ad[j - 1].wait()
            if j + 1 < K:
                gd[j + 1] = pltpu.async_copy(
                    x_hbm.at[rowb.at[pl.ds((j + 1) * G, G)]],
                    rows[(j + 1) % 2], gsem[(j + 1) % 2])
                _cole(j + 1)
            gd[j].wait()
            ad[j] = pltpu.async_copy(rows[j % 2], acc_sh.at[cole2d.at[j]],
                                     sem4, add=True)
        ad[K - 1].wait()
        return carry
    lax.fori_loop(0, NCHUNK // K, _macro, 0)

    plsc.subcore_barrier()

    pltpu.sync_copy(hist, cnt16_sh.at[s])
    plsc.subcore_barrier()

    pltpu.sync_copy(cnt16_sh.at[0, pl.ds(base_r, RPT)], redacc)
    for t in range(1, NS):
        pltpu.sync_copy(cnt16_sh.at[t, pl.ds(base_r, RPT)], redtmp)
        for q in range(RPT // L):
            redacc[pl.ds(q * L, L)] = (redacc[pl.ds(q * L, L)]
                                       + redtmp[pl.ds(q * L, L)])

    out_base = c * NP + base_r
    pltpu.sync_copy(acc_sh.at[pl.ds(base_r, RPT)],
                    acc_out.at[pl.ds(out_base, RPT)])
    pltpu.sync_copy(redacc, cnt_out.at[pl.ds(out_base, RPT)])


_sc_scatter = functools.partial(
    pl.kernel,
    out_type=(
        jax.ShapeDtypeStruct((NC * NP, D), jnp.bfloat16),
        jax.ShapeDtypeStruct((NC * NP,), jnp.float32),
    ),
    mesh=plsc.VectorSubcoreMesh(core_axis_name="c", subcore_axis_name="s"),
    scratch_types=[
        pltpu.VMEM_SHARED((NP, D), jnp.bfloat16),
        pltpu.VMEM_SHARED((NS, NP), jnp.float32),
        pltpu.VMEM((K * G,), jnp.int32),
        pltpu.VMEM((K * G,), jnp.int32),
        pltpu.VMEM((K, G), jnp.int32),
        pltpu.VMEM((G, D), jnp.bfloat16),
        pltpu.VMEM((G, D), jnp.bfloat16),
        pltpu.VMEM((NP,), jnp.float32),
        pltpu.VMEM((RPT,), jnp.float32),
        pltpu.VMEM((RPT,), jnp.float32),
        pltpu.VMEM((64, D), jnp.bfloat16),
        pltpu.SemaphoreType.DMA,
        pltpu.SemaphoreType.DMA,
        pltpu.SemaphoreType.DMA,
    ],
    compiler_params=pltpu.CompilerParams(use_tc_tiling_on_sc=False,
                                         needs_layout_passes=False),
)(_sc_scatter_kernel)


def _tc_finish_kernel(acc_ref, cnt_ref, x_ref, w_ref, b_ref, g_ref, be_ref,
                      o_ref):
    acc = acc_ref[...]
    cnt = cnt_ref[...]
    x = x_ref[...]
    accf = acc.astype(jnp.float32)
    s_tot = accf[0:N_NODES] + accf[NP:NP + N_NODES] + x
    c_tot = (cnt[0:N_NODES] + cnt[NP:NP + N_NODES] + 1.0).reshape(N_NODES, 1)
    aggr = s_tot / c_tot
    h = lax.dot_general(aggr, w_ref[...], (((1,), (1,)), ((), ())),
                        preferred_element_type=jnp.float32,
                        precision=lax.Precision.HIGHEST)
    h = h + b_ref[...]
    mean = jnp.mean(h, axis=0, keepdims=True)
    var = jnp.mean(jnp.square(h - mean), axis=0, keepdims=True)
    out = (h - mean) * lax.rsqrt(var + 1e-5) * g_ref[...] + be_ref[...] + x
    o_ref[...] = jnp.maximum(out, 0.0)


def _tc_finish(acc, cnt, x, W_lin, b_lin, gamma2, beta2):
    return pl.pallas_call(
        _tc_finish_kernel,
        out_shape=jax.ShapeDtypeStruct((N_NODES, D), jnp.float32),
    )(acc, cnt, x, W_lin, b_lin, gamma2, beta2)


def kernel(x, edge_index, W_lin, b_lin, gamma2, beta2):
    acc, cnt = _sc_scatter(edge_index, x.astype(jnp.bfloat16))
    return _tc_finish(acc, cnt, x, W_lin,
                      b_lin.reshape(1, D), gamma2.reshape(1, D),
                      beta2.reshape(1, D))


# speedup vs baseline: 21.2439x; 1.2407x over previous
import functools

import jax
import jax.numpy as jnp
from jax import lax
from jax.experimental import pallas as pl
from jax.experimental.pallas import tpu as pltpu
from jax.experimental.pallas import tpu_sc as plsc

N_NODES = 10000
N_EDGES = 320000
D = 128

NC = 2
NS = 16
L = 16
NW = NC * NS
EPW = N_EDGES // NW
G = 80
NCHUNK = EPW // G
K = 25
NP = 10240
DUMMY = N_NODES
RPT = NP // NS


def _sc_scatter_kernel(edge_hbm, x_hbm, acc_out, cnt_out,
                       acc_sh, cnt16_sh, rowb, colb, cole2d,
                       rows0, rows1, rows2, hist, red16, redacc, zbuf,
                       sem, sem2, sem3, sem4):
    c = lax.axis_index("c")
    s = lax.axis_index("s")
    wid = c * NS + s

    def _fill_h(i, carry):
        for u in range(4):
            hist[pl.ds((i * 4 + u) * L, L)] = jnp.zeros((L,), jnp.float32)
        return carry
    lax.fori_loop(0, NP // (4 * L), _fill_h, 0)

    def _fill_z(i, carry):
        for j in range(D // (2 * L)):
            zbuf[i, pl.ds(j * 2 * L, 2 * L)] = jnp.zeros((2 * L,),
                                                         jnp.bfloat16)
        return carry
    lax.fori_loop(0, 64, _fill_z, 0)

    base_r = s * RPT
    for k in range(RPT // 64):
        pltpu.sync_copy(zbuf, acc_sh.at[pl.ds(base_r + k * 64, 64)])
    plsc.subcore_barrier()

    ebase = wid * EPW
    rows = (rows0, rows1, rows2)
    gsem = (sem, sem2, sem3)
    one_v = jnp.ones((L,), jnp.float32)

    def _cole(j):
        for q in range(G // L):
            rv = rowb[pl.ds(j * G + q * L, L)]
            cv = colb[pl.ds(j * G + q * L, L)]
            ce = jnp.where(rv == cv, jnp.int32(DUMMY), cv)
            cole2d[j, pl.ds(q * L, L)] = ce
            plsc.addupdate_scatter(hist, [ce], one_v)

    def _macro(m, carry):
        off = ebase + m * (K * G)
        ri = pltpu.async_copy(edge_hbm.at[0, pl.ds(off, K * G)], rowb, sem)
        ci = pltpu.async_copy(edge_hbm.at[1, pl.ds(off, K * G)], colb, sem2)
        ri.wait()
        ci.wait()
        gd = [None] * K
        ad = [None] * K
        gd[0] = pltpu.async_copy(x_hbm.at[rowb.at[pl.ds(0, G)]], rows[0],
                                 gsem[0])
        _cole(0)
        for j in range(K):
            if j >= 2:
                ad[j - 2].wait()
            if j + 1 < K:
                gd[j + 1] = pltpu.async_copy(
                    x_hbm.at[rowb.at[pl.ds((j + 1) * G, G)]],
                    rows[(j + 1) % 3], gsem[(j + 1) % 3])
                _cole(j + 1)
            gd[j].wait()
            ad[j] = pltpu.async_copy(rows[j % 3], acc_sh.at[cole2d.at[j]],
                                     sem4, add=True)
        ad[K - 2].wait()
        ad[K - 1].wait()
        return carry
    lax.fori_loop(0, NCHUNK // K, _macro, 0)

    plsc.subcore_barrier()

    pltpu.sync_copy(hist, cnt16_sh.at[s])
    plsc.subcore_barrier()

    rd = [pltpu.async_copy(cnt16_sh.at[t, pl.ds(base_r, RPT)],
                           red16.at[t], sem) for t in range(NS)]
    for t in range(NS):
        rd[t].wait()
    for q in range(RPT // L):
        v = red16[0, pl.ds(q * L, L)]
        for t in range(1, NS):
            v = v + red16[t, pl.ds(q * L, L)]
        redacc[pl.ds(q * L, L)] = v

    out_base = c * NP + base_r
    pltpu.sync_copy(acc_sh.at[pl.ds(base_r, RPT)],
                    acc_out.at[pl.ds(out_base, RPT)])
    pltpu.sync_copy(redacc, cnt_out.at[pl.ds(out_base, RPT)])


_sc_scatter = functools.partial(
    pl.kernel,
    out_type=(
        jax.ShapeDtypeStruct((NC * NP, D), jnp.bfloat16),
        jax.ShapeDtypeStruct((NC * NP,), jnp.float32),
    ),
    mesh=plsc.VectorSubcoreMesh(core_axis_name="c", subcore_axis_name="s"),
    scratch_types=[
        pltpu.VMEM_SHARED((NP, D), jnp.bfloat16),
        pltpu.VMEM_SHARED((NS, NP), jnp.float32),
        pltpu.VMEM((K * G,), jnp.int32),
        pltpu.VMEM((K * G,), jnp.int32),
        pltpu.VMEM((K, G), jnp.int32),
        pltpu.VMEM((G, D), jnp.bfloat16),
        pltpu.VMEM((G, D), jnp.bfloat16),
        pltpu.VMEM((G, D), jnp.bfloat16),
        pltpu.VMEM((NP,), jnp.float32),
        pltpu.VMEM((NS, RPT), jnp.float32),
        pltpu.VMEM((RPT,), jnp.float32),
        pltpu.VMEM((64, D), jnp.bfloat16),
        pltpu.SemaphoreType.DMA,
        pltpu.SemaphoreType.DMA,
        pltpu.SemaphoreType.DMA,
        pltpu.SemaphoreType.DMA,
    ],
    compiler_params=pltpu.CompilerParams(use_tc_tiling_on_sc=False,
                                         needs_layout_passes=False),
)(_sc_scatter_kernel)


def _tc_finish_kernel(acc_ref, cnt_ref, x_ref, w_ref, b_ref, g_ref, be_ref,
                      o_ref):
    acc = acc_ref[...]
    cnt = cnt_ref[...]
    x = x_ref[...]
    accf = acc.astype(jnp.float32)
    s_tot = accf[0:N_NODES] + accf[NP:NP + N_NODES] + x
    c_tot = (cnt[0:N_NODES] + cnt[NP:NP + N_NODES] + 1.0).reshape(N_NODES, 1)
    aggr = s_tot / c_tot
    h = lax.dot_general(aggr, w_ref[...], (((1,), (1,)), ((), ())),
                        preferred_element_type=jnp.float32,
                        precision=lax.Precision.HIGHEST)
    h = h + b_ref[...]
    mean = jnp.mean(h, axis=0, keepdims=True)
    var = jnp.mean(jnp.square(h - mean), axis=0, keepdims=True)
    out = (h - mean) * lax.rsqrt(var + 1e-5) * g_ref[...] + be_ref[...] + x
    o_ref[...] = jnp.maximum(out, 0.0)


def _tc_finish(acc, cnt, x, W_lin, b_lin, gamma2, beta2):
    return pl.pallas_call(
        _tc_finish_kernel,
        out_shape=jax.ShapeDtypeStruct((N_NODES, D), jnp.float32),
    )(acc, cnt, x, W_lin, b_lin, gamma2, beta2)


def kernel(x, edge_index, W_lin, b_lin, gamma2, beta2):
    acc, cnt = _sc_scatter(edge_index, x.astype(jnp.bfloat16))
    return _tc_finish(acc, cnt, x, W_lin,
                      b_lin.reshape(1, D), gamma2.reshape(1, D),
                      beta2.reshape(1, D))
